# baseline (speedup 1.0000x reference)
import functools

import jax
import jax.numpy as jnp
from jax import lax
from jax.experimental import pallas as pl
from jax.experimental.pallas import tpu as pltpu
from jax.experimental.pallas import tpu_sc as plsc

_BATCH = 16384
_D = 64
_MARGIN = 1.0
_NC = 2
_NS = 16
_NW = _NC * _NS
_BW = _BATCH // _NW

_NROW = 1000000
_NTC = (_NROW + 127) // 128
_TCW = (_NTC + _NW - 1) // _NW
_CAP = 24
_SCHUNK = 4096
_STAGE = 192

_ENT_SLOTS = 4 * _BATCH
_REL_SLOTS = 2 * _BATCH
_GENT_ROWS = _ENT_SLOTS + _NW
_GREL_ROWS = _REL_SLOTS + _NW


def _extract_column(slab, col, staging, srow):
    cols = jnp.full((16,), col, jnp.int32)
    for q in range(4):
        rows = lax.iota(jnp.int32, 16) + q * 16
        v = plsc.load_gather(slab, [rows, cols])
        staging[srow, pl.ds(q * 16, 16)] = v


def _sget(ref, i):
    return ref[pl.ds(i, 16)][0]


def _sset(ref, i, val):
    v = ref[pl.ds(i, 16)]
    ref[pl.ds(i, 16)] = jnp.where(lax.iota(jnp.int32, 16) == 0, val, v)


def _k1_body(ph, pr, pt, nh, nr, nt, ent_t, rel_t, gent, grel,
             scanbuf, mids, mslots, buckets, counts, wins, fids, fslots,
             slab0, slab1, slabL, staging, sidx16, sidx192, sbuild, srowref,
             sem, slabsem):
    wid = lax.axis_index("s") * _NC + lax.axis_index("c")
    tc0 = wid * _TCW
    ntc = jnp.minimum(_TCW, _NTC - tc0)
    slabs = (slab0, slab1)

    def do_table(table, arrays, gout, trash):
        def zc(i, c):
            counts[pl.ds(i * 16, 16)] = jnp.zeros((16,), jnp.int32)
            return c
        lax.fori_loop(0, (_TCW + 15) // 16, zc, 0)

        sidx16[pl.ds(0, 16)] = jnp.full((16,), trash, jnp.int32)

        def overflow_one(idv, slot):
            tcv = idv >> 7

            @pl.when(tcv < _NTC - 1)
            def of_full():
                st = pl.multiple_of(tcv * 128, 128)
                pltpu.sync_copy(table.at[pl.ds(0, _D), pl.ds(st, 128)],
                                slab0.at[pl.ds(0, _D), pl.ds(0, 128)])
                _extract_column(slab0, idv & 127, staging, 0)

            @pl.when(tcv >= _NTC - 1)
            def of_part():
                pltpu.sync_copy(
                    table.at[pl.ds(0, _D), pl.ds((_NTC - 1) * 128, 64)],
                    slabL)
                _extract_column(slabL, idv & 127, staging, 0)

            tr = jnp.full((16,), trash, jnp.int32)
            sidx16[pl.ds(0, 16)] = jnp.where(
                lax.iota(jnp.int32, 16) == 0, slot, tr)
            pltpu.async_copy(staging.at[pl.ds(0, 16)],
                             gout.at[sidx16], sem).wait()
            sidx16[pl.ds(0, 16)] = tr

        for src, base in arrays:
            for ch in range(_BATCH // _SCHUNK):
                pltpu.sync_copy(src.at[pl.ds(ch * _SCHUNK, _SCHUNK)], scanbuf)

                def scan_vreg(i, pos):
                    v = scanbuf[pl.ds(i * 16, 16)]
                    tcl = (v >> 7) - tc0
                    m = (tcl >= 0) & (tcl < ntc)
                    slots = lax.iota(jnp.int32, 16) + (base + ch * _SCHUNK
                                                       + i * 16)
                    plsc.store_compressed(mids.at[pl.ds(pos, 16)], v, mask=m)
                    plsc.store_compressed(mslots.at[pl.ds(pos, 16)], slots, mask=m)
                    n = plsc.all_reduce_population_count(m)
                    return pos + lax.squeeze(lax.slice(n, (0,), (1,)), (0,))

                nmatch = lax.fori_loop(0, _SCHUNK // 16, scan_vreg,
                                       jnp.int32(0))

                def append(j, c):
                    idv = _sget(mids, j)
                    slot = _sget(mslots, j)
                    tcl = (idv >> 7) - tc0
                    col = idv & 127
                    cnt = _sget(counts, tcl)

                    @pl.when(cnt < _CAP)
                    def do_append():
                        _sset(buckets, tcl * _CAP + cnt, col | (slot << 7))
                        _sset(counts, tcl, cnt + 1)

                    @pl.when(cnt >= _CAP)
                    def do_overflow():
                        overflow_one(idv, slot)

                    return c

                del append
                lanes = lax.iota(jnp.int32, 16)

                def vappend(i, fpos):
                    ids = mids[pl.ds(i * 16, 16)]
                    slots = mslots[pl.ds(i * 16, 16)]
                    tcl = (ids >> 7) - tc0
                    valid = lanes < (nmatch - i * 16)
                    tcl = jnp.where(valid, tcl, lanes)
                    plsc.store_scatter(wins, [tcl], lanes)
                    winner = plsc.load_gather(wins, [tcl]) == lanes
                    cnt = plsc.load_gather(counts, [tcl])
                    ok = winner & (cnt < _CAP) & valid
                    entry = (ids & 127) | (slots << 7)
                    plsc.store_scatter(buckets, [tcl * _CAP + cnt], entry,
                                       mask=ok)
                    plsc.store_scatter(counts, [tcl], cnt + 1, mask=ok)
                    fb = valid & (~ok)
                    plsc.store_compressed(fids.at[pl.ds(fpos, 16)], ids,
                                          mask=fb)
                    plsc.store_compressed(fslots.at[pl.ds(fpos, 16)], slots,
                                          mask=fb)
                    n = plsc.all_reduce_population_count(fb)
                    return fpos + lax.squeeze(lax.slice(n, (0,), (1,)), (0,))

                nfb = lax.fori_loop(0, lax.div(nmatch + 15, jnp.int32(16)),
                                    vappend, jnp.int32(0))

                def fappend(j, c):
                    idv = _sget(fids, j)
                    slot = _sget(fslots, j)
                    tcl = (idv >> 7) - tc0
                    col = idv & 127
                    cnt = _sget(counts, tcl)

                    @pl.when(cnt < _CAP)
                    def do_append():
                        _sset(buckets, tcl * _CAP + cnt, col | (slot << 7))
                        _sset(counts, tcl, cnt + 1)

                    @pl.when(cnt >= _CAP)
                    def do_overflow():
                        overflow_one(idv, slot)

                    return c

                lax.fori_loop(0, nfb, fappend, 0)

        tr16 = jnp.full((16,), trash, jnp.int32)

        def prefill_sbuild():
            def pf(i, c):
                sbuild[pl.ds(i * 16, 16)] = tr16
                return c
            lax.fori_loop(0, _STAGE // 16 + 1, pf, 0)

        def flush():
            def cp(i, c):
                sidx192[pl.ds(i * 16, 16)] = sbuild[pl.ds(i * 16, 16)]
                return c
            lax.fori_loop(0, _STAGE // 16, cp, 0)
            pltpu.async_copy(staging, gout.at[sidx192], sem).wait()
            prefill_sbuild()
            _sset(srowref, 0, 0)

        prefill_sbuild()
        _sset(srowref, 0, 0)

        def process_slab(buf, tcl, cbase):
            cnt = _sget(counts, tcl)
            srow = _sget(srowref, 0)

            def ext(j, c2):
                e = _sget(buckets, tcl * _CAP + j)
                _extract_column(buf, cbase + (e & 127), staging, srow + j)
                _sset(sbuild, srow + j, e >> 7)
                return c2

            lax.fori_loop(0, cnt, ext, 0)
            _sset(srowref, 0, srow + cnt)

            @pl.when(srow + cnt >= _STAGE - _CAP)
            def do_flush():
                flush()

        has_last = tc0 + ntc >= _NTC
        nfull = ntc - jnp.where(has_last, 1, 0)
        nsuper = lax.div(nfull + 3, jnp.int32(4))

        def super_start(s):
            return jnp.minimum(tc0 + s * 4, _NTC - 5)

        def fire(s, buf):
            st = pl.multiple_of(super_start(s) * 128, 128)
            return pltpu.async_copy(
                table.at[pl.ds(0, _D), pl.ds(st, 512)], buf, slabsem)

        fire(0, slab0).wait()

        def proc(s, c):
            parity = lax.rem(s, 2)

            def with_buf(buf, other):
                nxt = fire(jnp.minimum(s + 1, nsuper - 1), other)
                st_tc = super_start(s)
                for k in range(4):
                    tcl = s * 4 + k

                    @pl.when(tcl < nfull)
                    def one():
                        process_slab(buf, tcl, (tc0 + tcl - st_tc) * 128)

                nxt.wait()

            @pl.when(parity == 0)
            def even():
                with_buf(slab0, slab1)

            @pl.when(parity == 1)
            def odd():
                with_buf(slab1, slab0)

            return c

        lax.fori_loop(0, nsuper, proc, 0)

        @pl.when(has_last)
        def last_slab():
            pltpu.sync_copy(
                table.at[pl.ds(0, _D), pl.ds((_NTC - 1) * 128, 64)], slabL)
            process_slab(slabL, ntc - 1, 0)

        flush()

    do_table(ent_t, ((ph, 0), (pt, _BATCH), (nh, 2 * _BATCH),
                     (nt, 3 * _BATCH)), gent, _ENT_SLOTS + wid)
    do_table(rel_t, ((pr, 0), (nr, _BATCH)), grel, _REL_SLOTS + wid)


def _k2_distance_group(hbuf, rbuf, tbuf, g):
    rows = lax.iota(jnp.int32, 16) + g * 16
    acc = jnp.zeros((16,), jnp.float32)
    for d in range(_D):
        cols = jnp.full((16,), d, jnp.int32)
        hv = plsc.load_gather(hbuf, [rows, cols])
        rv = plsc.load_gather(rbuf, [rows, cols])
        tv = plsc.load_gather(tbuf, [rows, cols])
        acc = acc + jnp.abs(hv + rv - tv)
    return acc


def _k2_body(gent, grel, out,
             ph0, pr0, pt0, nh0, nr0, nt0,
             ph1, pr1, pt1, nh1, nr1, nt1,
             ostage, sem0, sem1):
    wid = lax.axis_index("s") * _NC + lax.axis_index("c")
    base = wid * _BW
    _CH = 64
    _NCH = _BW // _CH
    _NGC = _CH // 16
    set0 = (ph0, pr0, pt0, nh0, nr0, nt0)
    set1 = (ph1, pr1, pt1, nh1, nr1, nt1)

    def srcs(c):
        cb = pl.multiple_of(base + c * _CH, 64)
        return (gent.at[pl.ds(cb, _CH)],
                grel.at[pl.ds(cb, _CH)],
                gent.at[pl.ds(_BATCH + cb, _CH)],
                gent.at[pl.ds(2 * _BATCH + cb, _CH)],
                grel.at[pl.ds(_BATCH + cb, _CH)],
                gent.at[pl.ds(3 * _BATCH + cb, _CH)])

    def fire(c, st, sem):
        for s, b in zip(srcs(c), st):
            pltpu.async_copy(s, b, sem)

    def drain(c, st, sem):
        for s, b in zip(srcs(c), st):
            pltpu.make_async_copy(s, b, sem).wait()

    def compute(st, lacc):
        hb, rb, tb, nhb, nrb, ntb = st

        def grp(g, l):
            pd = _k2_distance_group(hb, rb, tb, g)
            nd = _k2_distance_group(nhb, nrb, ntb, g)
            return l + jnp.maximum(pd - nd + _MARGIN, 0.0)

        return lax.fori_loop(0, _NGC, grp, lacc)

    fire(0, set0, sem0)

    def chunk_pair(cp, lacc):
        c0 = cp * 2
        fire(c0 + 1, set1, sem1)
        drain(c0, set0, sem0)
        lacc = compute(set0, lacc)

        @pl.when(c0 + 2 < _NCH)
        def prefetch():
            fire(c0 + 2, set0, sem0)

        drain(c0 + 1, set1, sem1)
        return compute(set1, lacc)

    lacc = lax.fori_loop(0, _NCH // 2, chunk_pair,
                         jnp.zeros((16,), jnp.float32))
    ostage[...] = lacc
    pltpu.sync_copy(ostage, out.at[pl.ds(wid * 16, 16)])


@jax.jit
def _transe_loss(ph, pr, pt, nh, nr, nt, ent_t, rel_t):
    mesh = plsc.VectorSubcoreMesh(core_axis_name="c", subcore_axis_name="s")
    k1 = functools.partial(
        pl.kernel,
        out_type=(jax.ShapeDtypeStruct((_GENT_ROWS, 128), jnp.float32),
                  jax.ShapeDtypeStruct((_GREL_ROWS, 128), jnp.float32)),
        mesh=mesh,
        scratch_types=[
            pltpu.VMEM((_SCHUNK,), jnp.int32),
            pltpu.VMEM((_SCHUNK + 32,), jnp.int32),
            pltpu.VMEM((_SCHUNK + 32,), jnp.int32),
            pltpu.VMEM((_TCW * _CAP + 16,), jnp.int32),
            pltpu.VMEM((((_TCW + 15) // 16) * 16 + 16,), jnp.int32),
            pltpu.VMEM((256,), jnp.int32),
            pltpu.VMEM((_SCHUNK + 32,), jnp.int32),
            pltpu.VMEM((_SCHUNK + 32,), jnp.int32),
            pltpu.VMEM((_D, 512), jnp.float32),
            pltpu.VMEM((_D, 512), jnp.float32),
            pltpu.VMEM((_D, 64), jnp.float32),
            pltpu.VMEM((_STAGE, 128), jnp.float32),
            pltpu.VMEM((16,), jnp.int32),
            pltpu.VMEM((_STAGE,), jnp.int32),
            pltpu.VMEM((_STAGE + 32,), jnp.int32),
            pltpu.VMEM((16,), jnp.int32),
            pltpu.SemaphoreType.DMA,
            pltpu.SemaphoreType.DMA,
        ],
        compiler_params=pltpu.CompilerParams(needs_layout_passes=False),
    )(_k1_body)
    gent, grel = k1(ph, pr, pt, nh, nr, nt, ent_t, rel_t)

    k2 = functools.partial(
        pl.kernel,
        out_type=jax.ShapeDtypeStruct((_NW * 16,), jnp.float32),
        mesh=mesh,
        scratch_types=(
            [pltpu.VMEM((64, 128), jnp.float32)] * 12 +
            [pltpu.VMEM((16,), jnp.float32),
             pltpu.SemaphoreType.DMA, pltpu.SemaphoreType.DMA]
        ),
        compiler_params=pltpu.CompilerParams(needs_layout_passes=False),
    )(_k2_body)
    partial_sums = k2(gent, grel)
    return jnp.sum(partial_sums) * (1.0 / _BATCH)


def kernel(positive_triples, negative_triples, ent_embedding, rel_embedding):
    return _transe_loss(
        positive_triples[:, 0], positive_triples[:, 1], positive_triples[:, 2],
        negative_triples[:, 0], negative_triples[:, 1], negative_triples[:, 2],
        ent_embedding.T, rel_embedding.T)


# speedup vs baseline: 1.2375x; 1.2375x over previous
import functools

import jax
import jax.numpy as jnp
from jax import lax
from jax.experimental import pallas as pl
from jax.experimental.pallas import tpu as pltpu
from jax.experimental.pallas import tpu_sc as plsc

_BATCH = 16384
_D = 64
_MARGIN = 1.0
_NC = 2
_NS = 16
_NW = _NC * _NS
_BW = _BATCH // _NW

_NROW = 1000000
_NTC = (_NROW + 127) // 128
_TCW = (_NTC + _NW - 1) // _NW
_CAP = 24
_SCHUNK = 4096
_STAGE = 192

_ENT_SLOTS = 4 * _BATCH
_REL_SLOTS = 2 * _BATCH
_GENT_ROWS = _ENT_SLOTS + _NW
_GREL_ROWS = _REL_SLOTS + _NW


def _extract_column(slab, col, staging, srow):
    cols = jnp.full((16,), col, jnp.int32)
    for q in range(4):
        rows = lax.iota(jnp.int32, 16) + q * 16
        v = plsc.load_gather(slab, [rows, cols])
        staging[srow, pl.ds(q * 16, 16)] = v


def _sget(ref, i):
    return ref[pl.ds(i, 16)][0]


def _sset(ref, i, val):
    v = ref[pl.ds(i, 16)]
    ref[pl.ds(i, 16)] = jnp.where(lax.iota(jnp.int32, 16) == 0, val, v)


def _k1_body(ph, pr, pt, nh, nr, nt, ent_t, rel_t, gent, grel,
             scanbuf, mids, mslots, buckets, counts, wins, fids, fslots,
             slab0, slab1, slabL, staging, sidx16, sidx192, sbuild, srowref,
             sem, slabsem):
    wid = lax.axis_index("s") * _NC + lax.axis_index("c")
    tc0 = wid * _TCW
    ntc = jnp.minimum(_TCW, _NTC - tc0)
    slabs = (slab0, slab1)

    def do_table(table, arrays, gout, trash):
        def zc(i, c):
            counts[pl.ds(i * 16, 16)] = jnp.zeros((16,), jnp.int32)
            return c
        lax.fori_loop(0, (_TCW + 15) // 16, zc, 0)

        sidx16[pl.ds(0, 16)] = jnp.full((16,), trash, jnp.int32)

        def overflow_one(idv, slot):
            tcv = idv >> 7

            @pl.when(tcv < _NTC - 1)
            def of_full():
                st = pl.multiple_of(tcv * 128, 128)
                pltpu.sync_copy(table.at[pl.ds(0, _D), pl.ds(st, 128)],
                                slab0.at[pl.ds(0, _D), pl.ds(0, 128)])
                _extract_column(slab0, idv & 127, staging, 0)

            @pl.when(tcv >= _NTC - 1)
            def of_part():
                pltpu.sync_copy(
                    table.at[pl.ds(0, _D), pl.ds((_NTC - 1) * 128, 64)],
                    slabL)
                _extract_column(slabL, idv & 127, staging, 0)

            tr = jnp.full((16,), trash, jnp.int32)
            sidx16[pl.ds(0, 16)] = jnp.where(
                lax.iota(jnp.int32, 16) == 0, slot, tr)
            pltpu.async_copy(staging.at[pl.ds(0, 16)],
                             gout.at[sidx16], sem).wait()
            sidx16[pl.ds(0, 16)] = tr

        for src, base in arrays:
            for ch in range(_BATCH // _SCHUNK):
                pltpu.sync_copy(src.at[pl.ds(ch * _SCHUNK, _SCHUNK)], scanbuf)

                def scan_vreg(i, pos):
                    v = scanbuf[pl.ds(i * 16, 16)]
                    tcl = (v >> 7) - tc0
                    m = (tcl >= 0) & (tcl < ntc)
                    slots = lax.iota(jnp.int32, 16) + (base + ch * _SCHUNK
                                                       + i * 16)
                    plsc.store_compressed(mids.at[pl.ds(pos, 16)], v, mask=m)
                    plsc.store_compressed(mslots.at[pl.ds(pos, 16)], slots, mask=m)
                    n = plsc.all_reduce_population_count(m)
                    return pos + lax.squeeze(lax.slice(n, (0,), (1,)), (0,))

                nmatch = lax.fori_loop(0, _SCHUNK // 16, scan_vreg,
                                       jnp.int32(0))

                def append(j, c):
                    idv = _sget(mids, j)
                    slot = _sget(mslots, j)
                    tcl = (idv >> 7) - tc0
                    col = idv & 127
                    cnt = _sget(counts, tcl)

                    @pl.when(cnt < _CAP)
                    def do_append():
                        _sset(buckets, tcl * _CAP + cnt, col | (slot << 7))
                        _sset(counts, tcl, cnt + 1)

                    @pl.when(cnt >= _CAP)
                    def do_overflow():
                        overflow_one(idv, slot)

                    return c

                del append
                lanes = lax.iota(jnp.int32, 16)

                def vappend(i, fpos):
                    ids = mids[pl.ds(i * 16, 16)]
                    slots = mslots[pl.ds(i * 16, 16)]
                    tcl = (ids >> 7) - tc0
                    valid = lanes < (nmatch - i * 16)
                    tcl = jnp.where(valid, tcl, lanes)
                    plsc.store_scatter(wins, [tcl], lanes)
                    winner = plsc.load_gather(wins, [tcl]) == lanes
                    cnt = plsc.load_gather(counts, [tcl])
                    ok = winner & (cnt < _CAP) & valid
                    entry = (ids & 127) | (slots << 7)
                    plsc.store_scatter(buckets, [tcl * _CAP + cnt], entry,
                                       mask=ok)
                    plsc.store_scatter(counts, [tcl], cnt + 1, mask=ok)
                    fb = valid & (~ok)
                    plsc.store_compressed(fids.at[pl.ds(fpos, 16)], ids,
                                          mask=fb)
                    plsc.store_compressed(fslots.at[pl.ds(fpos, 16)], slots,
                                          mask=fb)
                    n = plsc.all_reduce_population_count(fb)
                    return fpos + lax.squeeze(lax.slice(n, (0,), (1,)), (0,))

                nfb = lax.fori_loop(0, lax.div(nmatch + 15, jnp.int32(16)),
                                    vappend, jnp.int32(0))

                def fappend(j, c):
                    idv = _sget(fids, j)
                    slot = _sget(fslots, j)
                    tcl = (idv >> 7) - tc0
                    col = idv & 127
                    cnt = _sget(counts, tcl)

                    @pl.when(cnt < _CAP)
                    def do_append():
                        _sset(buckets, tcl * _CAP + cnt, col | (slot << 7))
                        _sset(counts, tcl, cnt + 1)

                    @pl.when(cnt >= _CAP)
                    def do_overflow():
                        overflow_one(idv, slot)

                    return c

                lax.fori_loop(0, nfb, fappend, 0)

        tr16 = jnp.full((16,), trash, jnp.int32)

        def prefill_sbuild():
            def pf(i, c):
                sbuild[pl.ds(i * 16, 16)] = tr16
                return c
            lax.fori_loop(0, _STAGE // 16 + 1, pf, 0)

        def flush():
            def cp(i, c):
                sidx192[pl.ds(i * 16, 16)] = sbuild[pl.ds(i * 16, 16)]
                return c
            lax.fori_loop(0, _STAGE // 16, cp, 0)
            pltpu.async_copy(staging, gout.at[sidx192], sem).wait()
            prefill_sbuild()
            _sset(srowref, 0, 0)

        prefill_sbuild()
        _sset(srowref, 0, 0)

        def process_slab(buf, tcl, cbase):
            cnt = _sget(counts, tcl)
            srow = _sget(srowref, 0)

            def ext(j, c2):
                e = _sget(buckets, tcl * _CAP + j)
                _extract_column(buf, cbase + (e & 127), staging, srow + j)
                _sset(sbuild, srow + j, e >> 7)
                return c2

            lax.fori_loop(0, cnt, ext, 0)
            _sset(srowref, 0, srow + cnt)

            @pl.when(srow + cnt >= _STAGE - _CAP)
            def do_flush():
                flush()

        has_last = tc0 + ntc >= _NTC
        nfull = ntc - jnp.where(has_last, 1, 0)
        nsuper = lax.div(nfull + 3, jnp.int32(4))

        def super_start(s):
            return jnp.minimum(tc0 + s * 4, _NTC - 5)

        def fire(s, buf):
            st = pl.multiple_of(super_start(s) * 128, 128)
            return pltpu.async_copy(
                table.at[pl.ds(0, _D), pl.ds(st, 512)], buf, slabsem)

        fire(0, slab0).wait()

        def proc(s, c):
            parity = lax.rem(s, 2)

            def with_buf(buf, other):
                nxt = fire(jnp.minimum(s + 1, nsuper - 1), other)
                st_tc = super_start(s)
                for k in range(4):
                    tcl = s * 4 + k

                    @pl.when(tcl < nfull)
                    def one():
                        process_slab(buf, tcl, (tc0 + tcl - st_tc) * 128)

                nxt.wait()

            @pl.when(parity == 0)
            def even():
                with_buf(slab0, slab1)

            @pl.when(parity == 1)
            def odd():
                with_buf(slab1, slab0)

            return c

        lax.fori_loop(0, nsuper, proc, 0)

        @pl.when(has_last)
        def last_slab():
            pltpu.sync_copy(
                table.at[pl.ds(0, _D), pl.ds((_NTC - 1) * 128, 64)], slabL)
            process_slab(slabL, ntc - 1, 0)

        flush()

    do_table(ent_t, ((ph, 0), (pt, _BATCH), (nh, 2 * _BATCH),
                     (nt, 3 * _BATCH)), gent, _ENT_SLOTS + wid)
    do_table(rel_t, ((pr, 0), (nr, _BATCH)), grel, _REL_SLOTS + wid)


def _k2_distance_group(hbuf, rbuf, tbuf, g):
    rows = lax.iota(jnp.int32, 16) + g * 16
    acc = jnp.zeros((16,), jnp.float32)
    for d in range(_D):
        cols = jnp.full((16,), d, jnp.int32)
        hv = plsc.load_gather(hbuf, [rows, cols])
        rv = plsc.load_gather(rbuf, [rows, cols])
        tv = plsc.load_gather(tbuf, [rows, cols])
        acc = acc + jnp.abs(hv + rv - tv)
    return acc


def _k2_tc_body(ph_ref, pr_ref, pt_ref, nh_ref, nr_ref, nt_ref, o_ref):
    mask = lax.broadcasted_iota(jnp.int32, (_TCB, 128), 1) < _D
    pdiff = jnp.abs(ph_ref[...] + pr_ref[...] - pt_ref[...])
    ndiff = jnp.abs(nh_ref[...] + nr_ref[...] - nt_ref[...])
    pd = jnp.sum(jnp.where(mask, pdiff, 0.0), axis=1)
    nd = jnp.sum(jnp.where(mask, ndiff, 0.0), axis=1)
    s = jnp.sum(jnp.maximum(pd - nd + _MARGIN, 0.0))
    o_ref[...] = jnp.broadcast_to(s, (1, 8, 128))


_TCB = 512


@jax.jit
def _transe_loss(ph, pr, pt, nh, nr, nt, ent_t, rel_t):
    mesh = plsc.VectorSubcoreMesh(core_axis_name="c", subcore_axis_name="s")
    k1 = functools.partial(
        pl.kernel,
        out_type=(jax.ShapeDtypeStruct((_GENT_ROWS, 128), jnp.float32),
                  jax.ShapeDtypeStruct((_GREL_ROWS, 128), jnp.float32)),
        mesh=mesh,
        scratch_types=[
            pltpu.VMEM((_SCHUNK,), jnp.int32),
            pltpu.VMEM((_SCHUNK + 32,), jnp.int32),
            pltpu.VMEM((_SCHUNK + 32,), jnp.int32),
            pltpu.VMEM((_TCW * _CAP + 16,), jnp.int32),
            pltpu.VMEM((((_TCW + 15) // 16) * 16 + 16,), jnp.int32),
            pltpu.VMEM((256,), jnp.int32),
            pltpu.VMEM((_SCHUNK + 32,), jnp.int32),
            pltpu.VMEM((_SCHUNK + 32,), jnp.int32),
            pltpu.VMEM((_D, 512), jnp.float32),
            pltpu.VMEM((_D, 512), jnp.float32),
            pltpu.VMEM((_D, 64), jnp.float32),
            pltpu.VMEM((_STAGE, 128), jnp.float32),
            pltpu.VMEM((16,), jnp.int32),
            pltpu.VMEM((_STAGE,), jnp.int32),
            pltpu.VMEM((_STAGE + 32,), jnp.int32),
            pltpu.VMEM((16,), jnp.int32),
            pltpu.SemaphoreType.DMA,
            pltpu.SemaphoreType.DMA,
        ],
        compiler_params=pltpu.CompilerParams(needs_layout_passes=False),
    )(_k1_body)
    gent, grel = k1(ph, pr, pt, nh, nr, nt, ent_t, rel_t)

    nblk = _BATCH // _TCB
    k2 = pl.pallas_call(
        _k2_tc_body,
        grid=(nblk,),
        in_specs=[
            pl.BlockSpec((_TCB, 128), lambda b: (b, 0)),
            pl.BlockSpec((_TCB, 128), lambda b: (b, 0)),
            pl.BlockSpec((_TCB, 128), lambda b: (b + nblk, 0)),
            pl.BlockSpec((_TCB, 128), lambda b: (b + 2 * nblk, 0)),
            pl.BlockSpec((_TCB, 128), lambda b: (b + nblk, 0)),
            pl.BlockSpec((_TCB, 128), lambda b: (b + 3 * nblk, 0)),
        ],
        out_specs=pl.BlockSpec((1, 8, 128), lambda b: (b, 0, 0)),
        out_shape=jax.ShapeDtypeStruct((nblk, 8, 128), jnp.float32),
    )
    partial_sums = k2(gent, grel, gent, gent, grel, gent)
    return jnp.sum(partial_sums[:, 0, 0]) * (1.0 / _BATCH)


def kernel(positive_triples, negative_triples, ent_embedding, rel_embedding):
    return _transe_loss(
        positive_triples[:, 0], positive_triples[:, 1], positive_triples[:, 2],
        negative_triples[:, 0], negative_triples[:, 1], negative_triples[:, 2],
        ent_embedding.T, rel_embedding.T)


# speedup vs baseline: 1.2862x; 1.0394x over previous
import functools

import jax
import jax.numpy as jnp
from jax import lax
from jax.experimental import pallas as pl
from jax.experimental.pallas import tpu as pltpu
from jax.experimental.pallas import tpu_sc as plsc

_BATCH = 16384
_D = 64
_MARGIN = 1.0
_NC = 2
_NS = 16
_NW = _NC * _NS
_BW = _BATCH // _NW

_NROW = 1000000
_NTC = (_NROW + 127) // 128
_TCW = (_NTC + _NW - 1) // _NW
_CAP = 24
_SCHUNK = 4096
_STAGE = 192

_ENT_SLOTS = 4 * _BATCH
_REL_SLOTS = 2 * _BATCH
_GENT_ROWS = _ENT_SLOTS + _NW
_GREL_ROWS = _REL_SLOTS + _NW


def _extract_column(slab, col, staging, srow):
    cols = jnp.full((16,), col, jnp.int32)
    for q in range(4):
        rows = lax.iota(jnp.int32, 16) + q * 16
        v = plsc.load_gather(slab, [rows, cols])
        staging[srow, pl.ds(q * 16, 16)] = v


def _sget(ref, i):
    return ref[pl.ds(i, 16)][0]


def _sset(ref, i, val):
    v = ref[pl.ds(i, 16)]
    ref[pl.ds(i, 16)] = jnp.where(lax.iota(jnp.int32, 16) == 0, val, v)


def _k1_body(ph, pr, pt, nh, nr, nt, ent_t, rel_t, gent, grel,
             scanbuf, scanbuf2, mids, mslots, buckets, counts, wins, fids,
             fslots,
             slab0, slab1, slabL, staging, sidx16, sidx192, sbuild, srowref,
             sem, slabsem, scansem):
    wid = lax.axis_index("s") * _NC + lax.axis_index("c")
    tc0 = wid * _TCW
    ntc = jnp.minimum(_TCW, _NTC - tc0)
    slabs = (slab0, slab1)

    def do_table(table, arrays, gout, trash):
        def zc(i, c):
            counts[pl.ds(i * 16, 16)] = jnp.zeros((16,), jnp.int32)
            return c
        lax.fori_loop(0, (_TCW + 15) // 16, zc, 0)

        sidx16[pl.ds(0, 16)] = jnp.full((16,), trash, jnp.int32)

        def overflow_one(idv, slot):
            tcv = idv >> 7

            @pl.when(tcv < _NTC - 1)
            def of_full():
                st = pl.multiple_of(tcv * 128, 128)
                pltpu.sync_copy(table.at[pl.ds(0, _D), pl.ds(st, 128)],
                                slab0.at[pl.ds(0, _D), pl.ds(0, 128)])
                _extract_column(slab0, idv & 127, staging, 0)

            @pl.when(tcv >= _NTC - 1)
            def of_part():
                pltpu.sync_copy(
                    table.at[pl.ds(0, _D), pl.ds((_NTC - 1) * 128, 64)],
                    slabL)
                _extract_column(slabL, idv & 127, staging, 0)

            tr = jnp.full((16,), trash, jnp.int32)
            sidx16[pl.ds(0, 16)] = jnp.where(
                lax.iota(jnp.int32, 16) == 0, slot, tr)
            pltpu.async_copy(staging.at[pl.ds(0, 16)],
                             gout.at[sidx16], sem).wait()
            sidx16[pl.ds(0, 16)] = tr

        chunks = [(s, b, ch) for s, b in arrays
                  for ch in range(_BATCH // _SCHUNK)]
        sbufs = (scanbuf, scanbuf2)

        def fire_chunk(k, buf):
            s, b, ch = chunks[k]
            return pltpu.async_copy(
                s.at[pl.ds(ch * _SCHUNK, _SCHUNK)], buf, scansem)

        pend = fire_chunk(0, sbufs[0])
        for k, (s, base, ch) in enumerate(chunks):
            pend.wait()
            buf = sbufs[k % 2]
            if k + 1 < len(chunks):
                pend = fire_chunk(k + 1, sbufs[(k + 1) % 2])

            def scan_vreg(i, pos, buf=buf, base=base, ch=ch):
                v = buf[pl.ds(i * 16, 16)]
                tcl = (v >> 7) - tc0
                m = (tcl >= 0) & (tcl < ntc)
                slots = lax.iota(jnp.int32, 16) + (base + ch * _SCHUNK
                                                   + i * 16)
                plsc.store_compressed(mids.at[pl.ds(pos, 16)], v, mask=m)
                plsc.store_compressed(mslots.at[pl.ds(pos, 16)], slots,
                                      mask=m)
                n = plsc.all_reduce_population_count(m)
                return pos + lax.squeeze(lax.slice(n, (0,), (1,)), (0,))

            nmatch = lax.fori_loop(0, _SCHUNK // 16, scan_vreg, jnp.int32(0))

            lanes = lax.iota(jnp.int32, 16)

            def vappend(i, fpos):
                ids = mids[pl.ds(i * 16, 16)]
                slots = mslots[pl.ds(i * 16, 16)]
                tcl = (ids >> 7) - tc0
                valid = lanes < (nmatch - i * 16)
                tcl = jnp.where(valid, tcl, lanes)
                plsc.store_scatter(wins, [tcl], lanes)
                winner = plsc.load_gather(wins, [tcl]) == lanes
                cnt = plsc.load_gather(counts, [tcl])
                ok = winner & (cnt < _CAP) & valid
                entry = (ids & 127) | (slots << 7)
                plsc.store_scatter(buckets, [tcl * _CAP + cnt], entry,
                                   mask=ok)
                plsc.store_scatter(counts, [tcl], cnt + 1, mask=ok)
                fb = valid & (~ok)
                plsc.store_compressed(fids.at[pl.ds(fpos, 16)], ids,
                                      mask=fb)
                plsc.store_compressed(fslots.at[pl.ds(fpos, 16)], slots,
                                      mask=fb)
                n = plsc.all_reduce_population_count(fb)
                return fpos + lax.squeeze(lax.slice(n, (0,), (1,)), (0,))

            nfb = lax.fori_loop(0, lax.div(nmatch + 15, jnp.int32(16)),
                                vappend, jnp.int32(0))

            def fappend(j, c):
                idv = _sget(fids, j)
                slot = _sget(fslots, j)
                tcl = (idv >> 7) - tc0
                col = idv & 127
                cnt = _sget(counts, tcl)

                @pl.when(cnt < _CAP)
                def do_append():
                    _sset(buckets, tcl * _CAP + cnt, col | (slot << 7))
                    _sset(counts, tcl, cnt + 1)

                @pl.when(cnt >= _CAP)
                def do_overflow():
                    overflow_one(idv, slot)

                return c

            lax.fori_loop(0, nfb, fappend, 0)

        tr16 = jnp.full((16,), trash, jnp.int32)

        def prefill_sbuild():
            def pf(i, c):
                sbuild[pl.ds(i * 16, 16)] = tr16
                return c
            lax.fori_loop(0, _STAGE // 16 + 1, pf, 0)

        def flush():
            def cp(i, c):
                sidx192[pl.ds(i * 16, 16)] = sbuild[pl.ds(i * 16, 16)]
                return c
            lax.fori_loop(0, _STAGE // 16, cp, 0)
            pltpu.async_copy(staging, gout.at[sidx192], sem).wait()
            prefill_sbuild()
            _sset(srowref, 0, 0)

        prefill_sbuild()
        _sset(srowref, 0, 0)

        def process_slab(buf, tcl, cbase):
            cnt = _sget(counts, tcl)
            srow = _sget(srowref, 0)

            def ext(j, c2):
                e = _sget(buckets, tcl * _CAP + j)
                _extract_column(buf, cbase + (e & 127), staging, srow + j)
                _sset(sbuild, srow + j, e >> 7)
                return c2

            lax.fori_loop(0, cnt, ext, 0)
            _sset(srowref, 0, srow + cnt)

            @pl.when(srow + cnt >= _STAGE - _CAP)
            def do_flush():
                flush()

        has_last = tc0 + ntc >= _NTC
        nfull = ntc - jnp.where(has_last, 1, 0)
        nsuper = lax.div(nfull + 3, jnp.int32(4))

        def super_start(s):
            return jnp.minimum(tc0 + s * 4, _NTC - 5)

        def fire(s, buf):
            st = pl.multiple_of(super_start(s) * 128, 128)
            return pltpu.async_copy(
                table.at[pl.ds(0, _D), pl.ds(st, 512)], buf, slabsem)

        fire(0, slab0).wait()

        def proc(s, c):
            parity = lax.rem(s, 2)

            def with_buf(buf, other):
                nxt = fire(jnp.minimum(s + 1, nsuper - 1), other)
                st_tc = super_start(s)
                for k in range(4):
                    tcl = s * 4 + k

                    @pl.when(tcl < nfull)
                    def one():
                        process_slab(buf, tcl, (tc0 + tcl - st_tc) * 128)

                nxt.wait()

            @pl.when(parity == 0)
            def even():
                with_buf(slab0, slab1)

            @pl.when(parity == 1)
            def odd():
                with_buf(slab1, slab0)

            return c

        lax.fori_loop(0, nsuper, proc, 0)

        @pl.when(has_last)
        def last_slab():
            pltpu.sync_copy(
                table.at[pl.ds(0, _D), pl.ds((_NTC - 1) * 128, 64)], slabL)
            process_slab(slabL, ntc - 1, 0)

        flush()

    do_table(ent_t, ((ph, 0), (pt, _BATCH), (nh, 2 * _BATCH),
                     (nt, 3 * _BATCH)), gent, _ENT_SLOTS + wid)
    do_table(rel_t, ((pr, 0), (nr, _BATCH)), grel, _REL_SLOTS + wid)


def _k2_distance_group(hbuf, rbuf, tbuf, g):
    rows = lax.iota(jnp.int32, 16) + g * 16
    acc = jnp.zeros((16,), jnp.float32)
    for d in range(_D):
        cols = jnp.full((16,), d, jnp.int32)
        hv = plsc.load_gather(hbuf, [rows, cols])
        rv = plsc.load_gather(rbuf, [rows, cols])
        tv = plsc.load_gather(tbuf, [rows, cols])
        acc = acc + jnp.abs(hv + rv - tv)
    return acc


def _k2_tc_body(ph_ref, pr_ref, pt_ref, nh_ref, nr_ref, nt_ref, o_ref):
    mask = lax.broadcasted_iota(jnp.int32, (_TCB, 128), 1) < _D
    pdiff = jnp.abs(ph_ref[...] + pr_ref[...] - pt_ref[...])
    ndiff = jnp.abs(nh_ref[...] + nr_ref[...] - nt_ref[...])
    pd = jnp.sum(jnp.where(mask, pdiff, 0.0), axis=1)
    nd = jnp.sum(jnp.where(mask, ndiff, 0.0), axis=1)
    s = jnp.sum(jnp.maximum(pd - nd + _MARGIN, 0.0))
    o_ref[...] = jnp.broadcast_to(s, (1, 8, 128))


_TCB = 512


@jax.jit
def _transe_loss(ph, pr, pt, nh, nr, nt, ent_t, rel_t):
    mesh = plsc.VectorSubcoreMesh(core_axis_name="c", subcore_axis_name="s")
    k1 = functools.partial(
        pl.kernel,
        out_type=(jax.ShapeDtypeStruct((_GENT_ROWS, 128), jnp.float32),
                  jax.ShapeDtypeStruct((_GREL_ROWS, 128), jnp.float32)),
        mesh=mesh,
        scratch_types=[
            pltpu.VMEM((_SCHUNK,), jnp.int32),
            pltpu.VMEM((_SCHUNK,), jnp.int32),
            pltpu.VMEM((_SCHUNK + 32,), jnp.int32),
            pltpu.VMEM((_SCHUNK + 32,), jnp.int32),
            pltpu.VMEM((_TCW * _CAP + 16,), jnp.int32),
            pltpu.VMEM((((_TCW + 15) // 16) * 16 + 16,), jnp.int32),
            pltpu.VMEM((256,), jnp.int32),
            pltpu.VMEM((_SCHUNK + 32,), jnp.int32),
            pltpu.VMEM((_SCHUNK + 32,), jnp.int32),
            pltpu.VMEM((_D, 512), jnp.float32),
            pltpu.VMEM((_D, 512), jnp.float32),
            pltpu.VMEM((_D, 64), jnp.float32),
            pltpu.VMEM((_STAGE, 128), jnp.float32),
            pltpu.VMEM((16,), jnp.int32),
            pltpu.VMEM((_STAGE,), jnp.int32),
            pltpu.VMEM((_STAGE + 32,), jnp.int32),
            pltpu.VMEM((16,), jnp.int32),
            pltpu.SemaphoreType.DMA,
            pltpu.SemaphoreType.DMA,
            pltpu.SemaphoreType.DMA,
        ],
        compiler_params=pltpu.CompilerParams(needs_layout_passes=False),
    )(_k1_body)
    gent, grel = k1(ph, pr, pt, nh, nr, nt, ent_t, rel_t)

    nblk = _BATCH // _TCB
    k2 = pl.pallas_call(
        _k2_tc_body,
        grid=(nblk,),
        in_specs=[
            pl.BlockSpec((_TCB, 128), lambda b: (b, 0)),
            pl.BlockSpec((_TCB, 128), lambda b: (b, 0)),
            pl.BlockSpec((_TCB, 128), lambda b: (b + nblk, 0)),
            pl.BlockSpec((_TCB, 128), lambda b: (b + 2 * nblk, 0)),
            pl.BlockSpec((_TCB, 128), lambda b: (b + nblk, 0)),
            pl.BlockSpec((_TCB, 128), lambda b: (b + 3 * nblk, 0)),
        ],
        out_specs=pl.BlockSpec((1, 8, 128), lambda b: (b, 0, 0)),
        out_shape=jax.ShapeDtypeStruct((nblk, 8, 128), jnp.float32),
    )
    partial_sums = k2(gent, grel, gent, gent, grel, gent)
    return jnp.sum(partial_sums[:, 0, 0]) * (1.0 / _BATCH)


def kernel(positive_triples, negative_triples, ent_embedding, rel_embedding):
    return _transe_loss(
        positive_triples[:, 0], positive_triples[:, 1], positive_triples[:, 2],
        negative_triples[:, 0], negative_triples[:, 1], negative_triples[:, 2],
        ent_embedding.T, rel_embedding.T)
